# baseline (speedup 1.0000x reference)
import functools

import jax
import jax.numpy as jnp
from jax import lax
from jax.experimental import pallas as pl
from jax.experimental.pallas import tpu as pltpu
from jax.experimental.pallas import tpu_sc as plsc

EPS_K = 0.1
L = 16
FILL_UNROLL = 10


def kernel(pred, target):
    n, c = pred.shape
    info = plsc.get_sparse_core_info()
    nc, ns = info.num_cores, info.num_subcores
    rows_per_core = n // nc
    blk_rows = 4
    chunks_per_core = rows_per_core // blk_rows
    chunks_per_sub = chunks_per_core // ns
    rows_per_sub = chunks_per_sub * blk_rows
    smooth = EPS_K / (c - 1)
    hot = 1.0 - EPS_K

    mesh = plsc.VectorSubcoreMesh(core_axis_name="c", subcore_axis_name="s")

    @functools.partial(
        pl.kernel,
        out_type=jax.ShapeDtypeStruct((n * c,), jnp.float32),
        mesh=mesh,
        scratch_types=[
            pltpu.VMEM((c,), jnp.float32),
            pltpu.VMEM_SHARED((blk_rows * c,), jnp.float32),
            pltpu.VMEM((rows_per_sub,), jnp.int32),
            pltpu.VMEM((rows_per_sub,), jnp.int32),
            pltpu.VMEM((rows_per_sub,), jnp.float32),
            pltpu.SemaphoreType.DMA,
            pltpu.SemaphoreType.DMA,
        ],
    )
    def sc_kernel(
        tgt_hbm, out_hbm, row_v, shared_v, tgt_v, eidx_v, hot_v, sem_fill, sem_hot
    ):
        cid = lax.axis_index("c")
        sid = lax.axis_index("s")
        smoothv = jnp.full((L,), smooth, jnp.float32)

        def fill_body(i, carry):
            base = pl.multiple_of(i * (L * FILL_UNROLL), L * FILL_UNROLL)
            for j in range(FILL_UNROLL):
                row_v[pl.ds(base + j * L, L)] = smoothv
            return carry

        lax.fori_loop(0, c // (L * FILL_UNROLL), fill_body, 0)

        @pl.when(sid < blk_rows)
        def _():
            pltpu.sync_copy(row_v, shared_v.at[pl.ds(sid * c, c)])

        plsc.subcore_barrier()

        n_groups = rows_per_sub // 8
        fills = []
        for g in range(n_groups):
            r0 = cid * rows_per_core + 8 * sid + 8 * ns * g
            for q in range(8 // blk_rows):
                dst0 = (r0 + q * blk_rows) * c
                fills.append(
                    pltpu.async_copy(
                        shared_v,
                        out_hbm.at[pl.ds(dst0, blk_rows * c)],
                        sem_fill,
                    )
                )

        iota = lax.iota(jnp.int32, L)
        for g in range(n_groups):
            r0 = cid * rows_per_core + 8 * sid + 8 * ns * g
            pltpu.sync_copy(
                tgt_hbm.at[pl.ds(r0, 8)],
                tgt_v.at[pl.ds(g * 8, 8)],
            )
        for g in range(0, n_groups, 2):
            ra = cid * rows_per_core + 8 * sid + 8 * ns * g
            rb = ra + 8 * ns
            rows16 = jnp.where(iota < 8, ra + iota, rb + (iota - 8))
            t = tgt_v[pl.ds(g * 8, L)]
            eidx_v[pl.ds(g * 8, L)] = rows16 * c + t
            hot_v[pl.ds(g * 8, L)] = jnp.full((L,), hot, jnp.float32)

        for d in fills:
            d.wait()

        pltpu.async_copy(hot_v, out_hbm.at[eidx_v], sem_hot).wait()

    out = sc_kernel(target.astype(jnp.int32))
    return out.reshape(n, c)
